# baseline (speedup 1.0000x reference)
import jax
import jax.numpy as jnp
from jax.experimental import pallas as pl
from jax.experimental.pallas import tpu as pltpu


def _combine_body(gw_ref, ef_ref, eb_ref, wflat_ref, b_ref):
    gates = jax.nn.softmax(gw_ref[...], axis=-1)
    wflat_ref[...] = jax.lax.dot_general(
        gates, ef_ref[...], (((1,), (0,)), ((), ())),
        precision=jax.lax.Precision.HIGHEST,
        preferred_element_type=jnp.float32)
    b_ref[...] = jax.lax.dot_general(
        gates, eb_ref[...], (((1,), (0,)), ((), ())),
        precision=jax.lax.Precision.HIGHEST,
        preferred_element_type=jnp.float32)


def _make_main_body(n_channels, batch, split):
    sub = batch // split

    def body(x_hbm, w_ref, b_ref, out_hbm,
             xbuf, obuf, x_sems, o_sems):
        n = pl.program_id(0)
        slot = jax.lax.rem(n, 2)
        nslot = jax.lax.rem(n + 1, 2)

        def x_start(idx, s):
            for k in range(split):
                rows = pl.ds(k * sub, sub)
                pltpu.make_async_copy(
                    x_hbm.at[rows, idx, :], xbuf.at[s, rows, :],
                    x_sems.at[s]).start()

        def x_wait(idx, s):
            for k in range(split):
                rows = pl.ds(k * sub, sub)
                pltpu.make_async_copy(
                    x_hbm.at[rows, idx, :], xbuf.at[s, rows, :],
                    x_sems.at[s]).wait()

        def o_start(idx, s):
            for k in range(split):
                rows = pl.ds(k * sub, sub)
                pltpu.make_async_copy(
                    obuf.at[s, rows, :], out_hbm.at[rows, idx, :],
                    o_sems.at[s]).start()

        def o_wait(idx, s):
            for k in range(split):
                rows = pl.ds(k * sub, sub)
                pltpu.make_async_copy(
                    obuf.at[s, rows, :], out_hbm.at[rows, idx, :],
                    o_sems.at[s]).wait()

        @pl.when(n == 0)
        def _():
            x_start(0, 0)

        @pl.when(n + 1 < n_channels)
        def _():
            x_start(n + 1, nslot)

        x_wait(n, slot)

        @pl.when(n >= 2)
        def _():
            o_wait(n - 2, slot)

        xs = xbuf[slot].astype(jnp.bfloat16)
        acc = jax.lax.dot_general(
            xs, w_ref[0], (((1,), (0,)), ((), ())),
            preferred_element_type=jnp.float32)
        obuf[slot] = acc + b_ref[0]
        o_start(n, slot)

        @pl.when(n == n_channels - 1)
        def _():
            o_wait(n - 1, nslot)
            o_wait(n, slot)
    return body


def kernel(x, gate_weights, experts, expert_biases):
    B, N, I = x.shape
    E, _, O = experts.shape

    experts_flat = experts.reshape(E, I * O)

    wflat, combined_b = pl.pallas_call(
        _combine_body,
        out_shape=[
            jax.ShapeDtypeStruct((N, I * O), jnp.float32),
            jax.ShapeDtypeStruct((N, O), jnp.float32),
        ],
    )(gate_weights, experts_flat, expert_biases)

    combined_w = wflat.reshape(N, I, O).astype(jnp.bfloat16)
    combined_b3 = combined_b.reshape(N, 1, O)

    out = pl.pallas_call(
        _make_main_body(N, B, 8),
        grid=(N,),
        in_specs=[
            pl.BlockSpec(memory_space=pltpu.MemorySpace.HBM),
            pl.BlockSpec((1, I, O), lambda n: (n, 0, 0)),
            pl.BlockSpec((1, 1, O), lambda n: (n, 0, 0)),
        ],
        out_specs=pl.BlockSpec(memory_space=pltpu.MemorySpace.HBM),
        out_shape=jax.ShapeDtypeStruct((B, N, O), jnp.float32),
        scratch_shapes=[
            pltpu.VMEM((2, B, I), jnp.float32),
            pltpu.VMEM((2, B, O), jnp.float32),
            pltpu.SemaphoreType.DMA((2,)),
            pltpu.SemaphoreType.DMA((2,)),
        ],
        compiler_params=pltpu.CompilerParams(
            dimension_semantics=("arbitrary",)),
    )(x, combined_w, combined_b3)

    return out


# speedup vs baseline: 1.0218x; 1.0218x over previous
import jax
import jax.numpy as jnp
from jax.experimental import pallas as pl
from jax.experimental.pallas import tpu as pltpu


def _combine_body(gw_ref, ef_ref, eb_ref, wflat_ref, b_ref):
    gates = jax.nn.softmax(gw_ref[...], axis=-1)
    wflat_ref[...] = jax.lax.dot_general(
        gates, ef_ref[...], (((1,), (0,)), ((), ())),
        precision=jax.lax.Precision.HIGHEST,
        preferred_element_type=jnp.float32)
    b_ref[...] = jax.lax.dot_general(
        gates, eb_ref[...], (((1,), (0,)), ((), ())),
        precision=jax.lax.Precision.HIGHEST,
        preferred_element_type=jnp.float32)


def _make_transpose_body(n_channels):
    def body(x_ref, w_ref, b_ref, out_ref, xt_ref, ot_ref):
        xt_ref[...] = jnp.transpose(x_ref[...], (1, 0, 2)).astype(jnp.bfloat16)
        for n in range(n_channels):
            acc = jax.lax.dot_general(
                xt_ref[n], w_ref[n], (((1,), (0,)), ((), ())),
                preferred_element_type=jnp.float32)
            ot_ref[n] = acc + b_ref[n][None, :]
        out_ref[...] = jnp.transpose(ot_ref[...], (1, 0, 2))
    return body


def _make_main_body(n_channels, batch, split):
    sub = batch // split

    def body(x_hbm, w_ref, b_ref, out_hbm,
             xbuf, obuf, x_sems, o_sems):
        n = pl.program_id(0)
        slot = jax.lax.rem(n, 2)
        nslot = jax.lax.rem(n + 1, 2)

        def x_start(idx, s):
            for k in range(split):
                rows = pl.ds(k * sub, sub)
                pltpu.make_async_copy(
                    x_hbm.at[rows, idx, :], xbuf.at[s, rows, :],
                    x_sems.at[s]).start()

        def x_wait(idx, s):
            for k in range(split):
                rows = pl.ds(k * sub, sub)
                pltpu.make_async_copy(
                    x_hbm.at[rows, idx, :], xbuf.at[s, rows, :],
                    x_sems.at[s]).wait()

        def o_start(idx, s):
            for k in range(split):
                rows = pl.ds(k * sub, sub)
                pltpu.make_async_copy(
                    obuf.at[s, rows, :], out_hbm.at[rows, idx, :],
                    o_sems.at[s]).start()

        def o_wait(idx, s):
            for k in range(split):
                rows = pl.ds(k * sub, sub)
                pltpu.make_async_copy(
                    obuf.at[s, rows, :], out_hbm.at[rows, idx, :],
                    o_sems.at[s]).wait()

        @pl.when(n == 0)
        def _():
            x_start(0, 0)

        @pl.when(n + 1 < n_channels)
        def _():
            x_start(n + 1, nslot)

        x_wait(n, slot)

        @pl.when(n >= 2)
        def _():
            o_wait(n - 2, slot)

        xs = xbuf[slot].astype(jnp.bfloat16)
        acc = jax.lax.dot_general(
            xs, w_ref[0], (((1,), (0,)), ((), ())),
            preferred_element_type=jnp.float32)
        obuf[slot] = acc + b_ref[0]
        o_start(n, slot)

        @pl.when(n == n_channels - 1)
        def _():
            o_wait(n - 1, nslot)
            o_wait(n, slot)
    return body


def kernel(x, gate_weights, experts, expert_biases):
    B, N, I = x.shape
    E, _, O = experts.shape

    experts_flat = experts.reshape(E, I * O)

    wflat, combined_b = pl.pallas_call(
        _combine_body,
        out_shape=[
            jax.ShapeDtypeStruct((N, I * O), jnp.float32),
            jax.ShapeDtypeStruct((N, O), jnp.float32),
        ],
    )(gate_weights, experts_flat, expert_biases)

    combined_w = wflat.reshape(N, I, O).astype(jnp.bfloat16)
    combined_b3 = combined_b.reshape(N, 1, O)

    BT = 128
    out = pl.pallas_call(
        _make_transpose_body(N),
        grid=(B // BT,),
        in_specs=[
            pl.BlockSpec((BT, N, I), lambda i: (i, 0, 0)),
            pl.BlockSpec((N, I, O), lambda i: (0, 0, 0)),
            pl.BlockSpec((N, O), lambda i: (0, 0)),
        ],
        out_specs=pl.BlockSpec((BT, N, O), lambda i: (i, 0, 0)),
        out_shape=jax.ShapeDtypeStruct((B, N, O), jnp.float32),
        scratch_shapes=[
            pltpu.VMEM((N, BT, I), jnp.bfloat16),
            pltpu.VMEM((N, BT, O), jnp.float32),
        ],
    )(x, combined_w, combined_b)

    return out
